# Initial kernel scaffold; baseline (speedup 1.0000x reference)
import jax
import jax.numpy as jnp
from jax.experimental import pallas as pl


def kernel(token_ids, weight):
    raise NotImplementedError("write your pallas kernel here")


# speedup vs baseline: 1.3044x; 1.3044x over previous
import functools
import math

import jax
import jax.numpy as jnp
from jax import lax
from jax.experimental import pallas as pl
from jax.experimental.pallas import tpu as pltpu
from jax.experimental.pallas import tpu_sc as plsc

D_MODEL = 1024
SCALE = math.sqrt(D_MODEL)
LANES = 16

NUM_CORES = 2
NUM_SUBCORES = 16
NW = NUM_CORES * NUM_SUBCORES

B_TOTAL = 4 * 4096
BPW = B_TOTAL // NW
CHUNK = 32
NCHUNK = BPW // CHUNK


def _sc_body(ids_hbm, w_hbm, out_hbm, idx_v, buf0, buf1, sem0, sem1):
    wid = lax.axis_index("s") * NUM_CORES + lax.axis_index("c")
    base = wid * BPW

    pltpu.sync_copy(ids_hbm.at[pl.ds(base, BPW)], idx_v)

    bufs = (buf0, buf1)
    sems = (sem0, sem1)

    def gather(g):
        return pltpu.async_copy(
            w_hbm.at[idx_v.at[pl.ds(g * CHUNK, CHUNK)]],
            bufs[g % 2],
            sems[g % 2],
        )

    def scale(buf):
        def row(i, carry):
            for j in range(D_MODEL // LANES):
                sl = (i, pl.ds(j * LANES, LANES))
                buf[sl] = buf[sl] * SCALE
            return carry

        lax.fori_loop(0, CHUNK, row, 0)

    pending = gather(0)
    for g in range(NCHUNK):
        nxt = None
        if g + 1 < NCHUNK:
            nxt = gather(g + 1)
        pending.wait()
        buf = bufs[g % 2]
        scale(buf)
        pltpu.sync_copy(buf, out_hbm.at[pl.ds(base + g * CHUNK, CHUNK)])
        pending = nxt


@jax.jit
def _embed(ids_flat, weight):
    mesh = plsc.VectorSubcoreMesh(core_axis_name="c", subcore_axis_name="s")
    k = functools.partial(
        pl.kernel,
        out_type=jax.ShapeDtypeStruct((B_TOTAL, D_MODEL), jnp.float32),
        mesh=mesh,
        scratch_types=[
            pltpu.VMEM((BPW,), jnp.int32),
            pltpu.VMEM((CHUNK, D_MODEL), jnp.float32),
            pltpu.VMEM((CHUNK, D_MODEL), jnp.float32),
            pltpu.SemaphoreType.DMA,
            pltpu.SemaphoreType.DMA,
        ],
    )(_sc_body)
    return k(ids_flat, weight)


def kernel(token_ids, weight):
    ids_flat = token_ids.reshape(-1).astype(jnp.int32)
    out = _embed(ids_flat, weight)
    return out.reshape(token_ids.shape + (D_MODEL,))


# speedup vs baseline: 1.4822x; 1.1363x over previous
import functools
import math

import jax
import jax.numpy as jnp
from jax import lax
from jax.experimental import pallas as pl
from jax.experimental.pallas import tpu as pltpu
from jax.experimental.pallas import tpu_sc as plsc

D_MODEL = 1024
SCALE = math.sqrt(D_MODEL)
LANES = 16

NUM_CORES = 2
NUM_SUBCORES = 16
NW = NUM_CORES * NUM_SUBCORES

B_TOTAL = 4 * 4096
BPW = B_TOTAL // NW
CHUNK = 32
NCHUNK = BPW // CHUNK


NBUF = 3


def _sc_body(ids_hbm, w_hbm, out_hbm, idx_v,
             buf0, buf1, buf2, gs0, gs1, gs2, ss0, ss1, ss2):
    wid = lax.axis_index("s") * NUM_CORES + lax.axis_index("c")
    base = wid * BPW

    pltpu.sync_copy(ids_hbm.at[pl.ds(base, BPW)], idx_v)

    bufs = (buf0, buf1, buf2)
    gsems = (gs0, gs1, gs2)
    ssems = (ss0, ss1, ss2)

    def gather(g):
        return pltpu.async_copy(
            w_hbm.at[idx_v.at[pl.ds(g * CHUNK, CHUNK)]],
            bufs[g % NBUF],
            gsems[g % NBUF],
        )

    def store(g):
        return pltpu.async_copy(
            bufs[g % NBUF],
            out_hbm.at[pl.ds(base + g * CHUNK, CHUNK)],
            ssems[g % NBUF],
        )

    def scale(buf):
        def row(i, carry):
            for j in range(D_MODEL // LANES):
                sl = (i, pl.ds(j * LANES, LANES))
                buf[sl] = buf[sl] * SCALE
            return carry

        lax.fori_loop(0, CHUNK, row, 0)

    gathers = [None] * NCHUNK
    stores = [None] * NCHUNK
    gathers[0] = gather(0)
    gathers[1] = gather(1)
    for g in range(NCHUNK):
        gathers[g].wait()
        scale(bufs[g % NBUF])
        stores[g] = store(g)
        if g + 2 < NCHUNK:
            if g >= 1:
                stores[g - 1].wait()
            gathers[g + 2] = gather(g + 2)
    stores[NCHUNK - 2].wait()
    stores[NCHUNK - 1].wait()


@jax.jit
def _embed(ids_flat, weight):
    mesh = plsc.VectorSubcoreMesh(core_axis_name="c", subcore_axis_name="s")
    k = functools.partial(
        pl.kernel,
        out_type=jax.ShapeDtypeStruct((B_TOTAL, D_MODEL), jnp.float32),
        mesh=mesh,
        scratch_types=[
            pltpu.VMEM((BPW,), jnp.int32),
            pltpu.VMEM((CHUNK, D_MODEL), jnp.float32),
            pltpu.VMEM((CHUNK, D_MODEL), jnp.float32),
            pltpu.VMEM((CHUNK, D_MODEL), jnp.float32),
            pltpu.SemaphoreType.DMA,
            pltpu.SemaphoreType.DMA,
            pltpu.SemaphoreType.DMA,
            pltpu.SemaphoreType.DMA,
            pltpu.SemaphoreType.DMA,
            pltpu.SemaphoreType.DMA,
        ],
    )(_sc_body)
    return k(ids_flat, weight)


def kernel(token_ids, weight):
    ids_flat = token_ids.reshape(-1).astype(jnp.int32)
    out = _embed(ids_flat, weight)
    return out.reshape(token_ids.shape + (D_MODEL,))
